# baseline (speedup 1.0000x reference)
import jax
import jax.numpy as jnp
from jax.experimental import pallas as pl
from jax.experimental.pallas import tpu as pltpu

_TB = 4


def _body(s_ref, p_ref, o_ref):
    jt = pl.program_id(0)
    base = pl.num_programs(0) * _TB - 1 - jt * _TB
    for k in range(_TB):
        prow = p_ref[base - k]
        o_ref[k] = s_ref[k] + jnp.broadcast_to(prow, s_ref.shape[1:])


def kernel(sessions, pos_emb):
    B, T, F = sessions.shape
    st = jnp.transpose(sessions, (1, 2, 0))
    pos3 = pos_emb[:, :, None]
    out_t = pl.pallas_call(
        _body,
        grid=(T // _TB,),
        in_specs=[
            pl.BlockSpec((_TB, F, B), lambda jt: (jt, 0, 0)),
            pl.BlockSpec((T, F, 1), lambda jt: (0, 0, 0)),
        ],
        out_specs=pl.BlockSpec((_TB, F, B), lambda jt: (jt, 0, 0)),
        out_shape=jax.ShapeDtypeStruct((T, F, B), sessions.dtype),
        compiler_params=pltpu.CompilerParams(
            dimension_semantics=("arbitrary",),
        ),
    )(st, pos3)
    return jnp.transpose(out_t, (2, 0, 1))


# speedup vs baseline: 1.0166x; 1.0166x over previous
import jax
import jax.numpy as jnp
from jax.experimental import pallas as pl
from jax.experimental.pallas import tpu as pltpu

_TB = 10


def _body(s_ref, p_ref, o_ref):
    jt = pl.program_id(0)
    base = pl.num_programs(0) * _TB - 1 - jt * _TB
    for k in range(_TB):
        prow = p_ref[base - k]
        o_ref[k] = s_ref[k] + jnp.broadcast_to(prow, s_ref.shape[1:])


def kernel(sessions, pos_emb):
    B, T, F = sessions.shape
    st = jnp.transpose(sessions, (1, 2, 0))
    pos3 = pos_emb[:, :, None]
    out_t = pl.pallas_call(
        _body,
        grid=(T // _TB,),
        in_specs=[
            pl.BlockSpec((_TB, F, B), lambda jt: (jt, 0, 0)),
            pl.BlockSpec((T, F, 1), lambda jt: (0, 0, 0)),
        ],
        out_specs=pl.BlockSpec((_TB, F, B), lambda jt: (jt, 0, 0)),
        out_shape=jax.ShapeDtypeStruct((T, F, B), sessions.dtype),
        compiler_params=pltpu.CompilerParams(
            dimension_semantics=("arbitrary",),
        ),
    )(st, pos3)
    return jnp.transpose(out_t, (2, 0, 1))
